# Initial kernel scaffold; baseline (speedup 1.0000x reference)
import jax
import jax.numpy as jnp
from jax.experimental import pallas as pl


def kernel(x, edge_weight, W1, b1, W2, b2, W3, b3, g1, beta1, g2, beta2, dW1, db1, dW2, db2, dW3, db3, edge_index, edge_label_index_only, neg_edge_index):
    raise NotImplementedError("write your pallas kernel here")


# speedup vs baseline: 6.2118x; 6.2118x over previous
import jax
import jax.numpy as jnp
from jax import lax
from jax.experimental import pallas as pl
from jax.experimental.pallas import tpu as pltpu
from jax.experimental.pallas import tpu_sc as plsc

N = 10000
NPAD = 10240
E = 320000
D = 128
NP2 = 320000
EPS = 1e-5

NC = 2
NS = 16
NW = NC * NS

CH = 80
EPT = E // NW
NCH = EPT // CH
PPT = NP2 // NW
RPT = NPAD // NS

_f32 = jnp.float32
_i32 = jnp.int32


def _mesh():
    return plsc.VectorSubcoreMesh(core_axis_name="c", subcore_axis_name="s",
                                  num_cores=NC, num_subcores=NS)


def _deg_body(w, dst, zeros, out, wv, idxs, acc):
    cc = lax.axis_index("c")
    sid = lax.axis_index("s")
    wid = cc * NS + sid

    @pl.when(sid == 0)
    def _():
        pltpu.sync_copy(zeros, acc)
    plsc.subcore_barrier()

    def chunk(c, carry):
        off = wid * EPT + c * CH
        pltpu.sync_copy(w.at[pl.ds(off, CH)], wv)
        pltpu.sync_copy(dst.at[pl.ds(off, CH)], idxs)
        pltpu.sync_copy(wv, acc.at[idxs], add=True)
        return carry
    lax.fori_loop(0, NCH, chunk, 0)

    plsc.subcore_barrier()

    @pl.when(sid == 0)
    def _():
        pltpu.sync_copy(acc, out.at[cc])


def _sc_deg(w, dst, zeros):
    return pl.kernel(
        _deg_body,
        out_type=jax.ShapeDtypeStruct((NC, N), _f32),
        mesh=_mesh(),
        scratch_types=[
            pltpu.VMEM((CH,), _f32),
            pltpu.VMEM((CH,), _i32),
            pltpu.VMEM_SHARED((N,), _f32),
        ],
    )(w, dst, zeros)


def _mp_body(y, src, dst, w, out, idxg, idxs, wv, rows, acc, sem):
    cc = lax.axis_index("c")
    sid = lax.axis_index("s")
    wid = cc * NS + sid

    def zr(i, carry):
        for j in range(8):
            rows[i, pl.ds(j * 16, 16)] = jnp.zeros((16,), _f32)
        return carry
    lax.fori_loop(0, CH, zr, 0)

    base = sid * RPT
    for k in range(RPT // CH):
        pltpu.sync_copy(rows, acc.at[pl.ds(base + k * CH, CH)])
    plsc.subcore_barrier()

    def chunk(c, carry):
        off = wid * EPT + c * CH
        pltpu.sync_copy(src.at[pl.ds(off, CH)], idxg)
        pltpu.sync_copy(dst.at[pl.ds(off, CH)], idxs)
        pltpu.sync_copy(w.at[pl.ds(off, CH)], wv)
        pltpu.async_copy(y.at[idxg], rows, sem).wait()

        def group(g, c2):
            wvec = wv[pl.ds(g * 16, 16)]
            for lane in range(16):
                ws = wvec[lane]
                e = g * 16 + lane
                for j in range(8):
                    s = pl.ds(j * 16, 16)
                    rows[e, s] = rows[e, s] * ws
            return c2
        lax.fori_loop(0, CH // 16, group, 0)

        pltpu.sync_copy(rows, acc.at[idxs], add=True)
        return carry
    lax.fori_loop(0, NCH, chunk, 0)

    plsc.subcore_barrier()
    pltpu.sync_copy(acc.at[pl.ds(base, RPT)], out.at[cc, pl.ds(base, RPT)])


def _sc_mp(y, src, dst, w):
    return pl.kernel(
        _mp_body,
        out_type=jax.ShapeDtypeStruct((NC, NPAD, D), _f32),
        mesh=_mesh(),
        scratch_types=[
            pltpu.VMEM((CH,), _i32),
            pltpu.VMEM((CH,), _i32),
            pltpu.VMEM((CH,), _f32),
            pltpu.VMEM((CH, D), _f32),
            pltpu.VMEM_SHARED((NPAD, D), _f32),
            pltpu.SemaphoreType.DMA,
        ],
    )(y, src, dst, w)


def _dec_body(p, q, i0, i1, out, i0v, i1v, rowsa, rowsb, sema, semb):
    cc = lax.axis_index("c")
    sid = lax.axis_index("s")
    wid = cc * NS + sid

    def chunk(c, carry):
        off = wid * PPT + c * CH
        pltpu.sync_copy(i0.at[pl.ds(off, CH)], i0v)
        pltpu.sync_copy(i1.at[pl.ds(off, CH)], i1v)
        da = pltpu.async_copy(p.at[i0v], rowsa, sema)
        db = pltpu.async_copy(q.at[i1v], rowsb, semb)
        da.wait()
        db.wait()

        def pair(e, c2):
            for j in range(8):
                s = pl.ds(j * 16, 16)
                rowsa[e, s] = jnp.maximum(rowsa[e, s] + rowsb[e, s], 0.0)
            return c2
        lax.fori_loop(0, CH, pair, 0)

        pltpu.sync_copy(rowsa, out.at[pl.ds(off, CH)])
        return carry
    lax.fori_loop(0, PPT // CH, chunk, 0)


def _sc_dec(p, q, i0, i1):
    return pl.kernel(
        _dec_body,
        out_type=jax.ShapeDtypeStruct((NP2, D), _f32),
        mesh=_mesh(),
        scratch_types=[
            pltpu.VMEM((CH,), _i32),
            pltpu.VMEM((CH,), _i32),
            pltpu.VMEM((CH, D), _f32),
            pltpu.VMEM((CH, D), _f32),
            pltpu.SemaphoreType.DMA,
            pltpu.SemaphoreType.DMA,
        ],
    )(p, q, i0, i1)


def _tc_first_body(degp, x, w1, dis_o, y_o):
    dis = lax.rsqrt(1.0 + degp[0] + degp[1])
    dis_o[...] = dis
    xw = jnp.dot(x[...], w1[...], preferred_element_type=_f32)
    y_o[...] = xw * dis


def _tc_first(degp, x, w1):
    return pl.pallas_call(
        _tc_first_body,
        out_shape=(jax.ShapeDtypeStruct((NPAD, 1), _f32),
                   jax.ShapeDtypeStruct((NPAD, D), _f32)),
    )(degp, x, w1)


def _tc_mid_body(m, y, dis, b, g, beta, w, y_o):
    h = (m[0] + m[1] + y[...]) * dis[...] + b[...]
    hv = h[:N]
    mu = jnp.mean(hv, axis=0, keepdims=True)
    var = jnp.mean((hv - mu) ** 2, axis=0, keepdims=True)
    hn = (h - mu) * lax.rsqrt(var + EPS) * g[...] + beta[...]
    hn = jnp.maximum(hn, 0.0)
    y_o[...] = jnp.dot(hn, w[...], preferred_element_type=_f32) * dis[...]


def _tc_mid(m, y, dis, b, g, beta, w):
    return pl.pallas_call(
        _tc_mid_body,
        out_shape=jax.ShapeDtypeStruct((NPAD, D), _f32),
    )(m, y, dis, b, g, beta, w)


def _tc_enc_body(m, y, dis, b, dw1a, dw1b, db1, p_o, q_o):
    enc = (m[0] + m[1] + y[...]) * dis[...] + b[...]
    p_o[...] = jnp.dot(enc, dw1a[...], preferred_element_type=_f32) + db1[...]
    q_o[...] = jnp.dot(enc, dw1b[...], preferred_element_type=_f32)


def _tc_enc(m, y, dis, b, dw1a, dw1b, db1):
    return pl.pallas_call(
        _tc_enc_body,
        out_shape=(jax.ShapeDtypeStruct((NPAD, D), _f32),
                   jax.ShapeDtypeStruct((NPAD, D), _f32)),
    )(m, y, dis, b, dw1a, dw1b, db1)


_DEC_R = 1280


def _tc_dec_body(h1, dw2, db2, dw3, db3, o):
    t = jnp.dot(h1[...], dw2[...], preferred_element_type=_f32) + db2[...]
    t = jnp.maximum(t, 0.0)
    o[...] = jnp.dot(t, dw3[...], preferred_element_type=_f32) + db3[...]


def _tc_dec(h1, dw2, db2, dw3, db3):
    grid = NP2 // _DEC_R
    return pl.pallas_call(
        _tc_dec_body,
        grid=(grid,),
        in_specs=[
            pl.BlockSpec((_DEC_R, D), lambda i: (i, 0)),
            pl.BlockSpec((D, D), lambda i: (0, 0)),
            pl.BlockSpec((1, D), lambda i: (0, 0)),
            pl.BlockSpec((D, 1), lambda i: (0, 0)),
            pl.BlockSpec((1, 1), lambda i: (0, 0)),
        ],
        out_specs=pl.BlockSpec((_DEC_R, 1), lambda i: (i, 0)),
        out_shape=jax.ShapeDtypeStruct((NP2, 1), _f32),
    )(h1, dw2, db2, dw3, db3)


def kernel(x, edge_weight, W1, b1, W2, b2, W3, b3, g1, beta1, g2, beta2,
           dW1, db1, dW2, db2, dW3, db3, edge_index,
           edge_label_index_only, neg_edge_index):
    src = edge_index[0].astype(_i32)
    dst = edge_index[1].astype(_i32)
    i0 = jnp.concatenate(
        [edge_label_index_only[0], neg_edge_index[0]]).astype(_i32)
    i1 = jnp.concatenate(
        [edge_label_index_only[1], neg_edge_index[1]]).astype(_i32)

    xp = jnp.pad(x, ((0, NPAD - N), (0, 0)))

    degp = _sc_deg(edge_weight, dst, jnp.zeros((N,), _f32))
    degp = jnp.pad(degp, ((0, 0), (0, NPAD - N))).reshape(NC, NPAD, 1)
    dis, y1 = _tc_first(degp, xp, W1)

    m1 = _sc_mp(y1, src, dst, edge_weight)
    y2 = _tc_mid(m1, y1, dis, b1.reshape(1, D), g1.reshape(1, D),
                 beta1.reshape(1, D), W2)
    m2 = _sc_mp(y2, src, dst, edge_weight)
    y3 = _tc_mid(m2, y2, dis, b2.reshape(1, D), g2.reshape(1, D),
                 beta2.reshape(1, D), W3)
    m3 = _sc_mp(y3, src, dst, edge_weight)

    p, q = _tc_enc(m3, y3, dis, b3.reshape(1, D), dW1[:D], dW1[D:],
                   db1.reshape(1, D))
    h1 = _sc_dec(p, q, i0, i1)
    out = _tc_dec(h1, dW2, db2.reshape(1, D), dW3, db3.reshape(1, 1))
    return out[:, 0]
